# baseline (speedup 1.0000x reference)
import functools

import jax
import jax.numpy as jnp
from jax import lax
from jax.experimental import pallas as pl
from jax.experimental.pallas import tpu as pltpu
from jax.experimental.pallas import tpu_sc as plsc

_V = 100000
_H = 128
_VBLK = 8192


def _gather_rows(table, idx, n_tok):
    info = plsc.get_sparse_core_info()
    nw = info.num_cores * info.num_subcores
    per = n_tok // nw
    mesh = plsc.VectorSubcoreMesh(core_axis_name="c", subcore_axis_name="s")

    @functools.partial(
        pl.kernel,
        mesh=mesh,
        out_type=jax.ShapeDtypeStruct((n_tok, _H), jnp.float32),
        scratch_types=[
            pltpu.VMEM((per,), jnp.int32),
            pltpu.VMEM((per, _H), jnp.float32),
            pltpu.SemaphoreType.DMA,
        ],
    )
    def k(table_hbm, idx_hbm, out_hbm, idx_v, rows_v, sem):
        wid = lax.axis_index("s") * info.num_cores + lax.axis_index("c")
        base = wid * per
        pltpu.sync_copy(idx_hbm.at[pl.ds(base, per)], idx_v)
        pltpu.async_copy(table_hbm.at[idx_v], rows_v, sem).wait()
        pltpu.sync_copy(rows_v, out_hbm.at[pl.ds(base, per)])

    return k(table, idx)


def _mm_body(h_ref, w_ref, o_ref):
    o_ref[...] = lax.dot_general(
        h_ref[...].astype(jnp.bfloat16),
        w_ref[...],
        (((1,), (1,)), ((), ())),
        preferred_element_type=jnp.float32,
    )


def kernel(input_ids, wte, lm_head_w):
    b, s = input_ids.shape
    n_tok = b * s
    idx = input_ids.reshape(n_tok).astype(jnp.int32)
    hidden = _gather_rows(wte, idx, n_tok)
    w_bf = lm_head_w.astype(jnp.bfloat16)
    nblk = pl.cdiv(_V, _VBLK)
    logits = pl.pallas_call(
        _mm_body,
        grid=(nblk,),
        in_specs=[
            pl.BlockSpec((n_tok, _H), lambda i: (0, 0)),
            pl.BlockSpec((_VBLK, _H), lambda i: (i, 0)),
        ],
        out_specs=pl.BlockSpec((n_tok, _VBLK), lambda i: (0, i)),
        out_shape=jax.ShapeDtypeStruct((n_tok, _V), jnp.float32),
        compiler_params=pltpu.CompilerParams(
            dimension_semantics=("arbitrary",),
            allow_input_fusion=[False, True],
        ),
    )(hidden, w_bf)
    return logits.reshape(b, s, _V)


# speedup vs baseline: 1.0527x; 1.0527x over previous
import functools

import jax
import jax.numpy as jnp
from jax import lax
from jax.experimental import pallas as pl
from jax.experimental.pallas import tpu as pltpu
from jax.experimental.pallas import tpu_sc as plsc

_V = 100000
_H = 128
_VBLK = 8192


def _gather_rows(table, idx, n_tok):
    info = plsc.get_sparse_core_info()
    nw = info.num_cores * info.num_subcores
    per = n_tok // nw
    mesh = plsc.VectorSubcoreMesh(core_axis_name="c", subcore_axis_name="s")

    @functools.partial(
        pl.kernel,
        mesh=mesh,
        out_type=jax.ShapeDtypeStruct((n_tok, _H), jnp.float32),
        scratch_types=[
            pltpu.VMEM((per,), jnp.int32),
            pltpu.VMEM((per, _H), jnp.float32),
            pltpu.SemaphoreType.DMA,
        ],
    )
    def k(table_hbm, idx_hbm, out_hbm, idx_v, rows_v, sem):
        wid = lax.axis_index("s") * info.num_cores + lax.axis_index("c")
        base = wid * per
        pltpu.sync_copy(idx_hbm.at[pl.ds(base, per)], idx_v)
        pltpu.async_copy(table_hbm.at[idx_v], rows_v, sem).wait()
        pltpu.sync_copy(rows_v, out_hbm.at[pl.ds(base, per)])

    return k(table, idx)


def _mm_body(h_ref, w_ref, o_ref):
    o_ref[...] = lax.dot_general(
        h_ref[...],
        w_ref[...],
        (((1,), (1,)), ((), ())),
        preferred_element_type=jnp.float32,
    )


def kernel(input_ids, wte, lm_head_w):
    b, s = input_ids.shape
    n_tok = b * s
    idx = input_ids.reshape(n_tok).astype(jnp.int32)
    hidden = _gather_rows(wte, idx, n_tok)
    nblk = pl.cdiv(_V, _VBLK)
    logits = pl.pallas_call(
        _mm_body,
        grid=(nblk,),
        in_specs=[
            pl.BlockSpec((n_tok, _H), lambda i: (0, 0)),
            pl.BlockSpec((_VBLK, _H), lambda i: (i, 0)),
        ],
        out_specs=pl.BlockSpec((n_tok, _VBLK), lambda i: (0, i)),
        out_shape=jax.ShapeDtypeStruct((n_tok, _V), jnp.float32),
        compiler_params=pltpu.CompilerParams(
            dimension_semantics=("arbitrary",),
        ),
    )(hidden, lm_head_w)
    return logits.reshape(b, s, _V)


# speedup vs baseline: 1.0562x; 1.0033x over previous
import functools

import jax
import jax.numpy as jnp
from jax import lax
from jax.experimental import pallas as pl
from jax.experimental.pallas import tpu as pltpu
from jax.experimental.pallas import tpu_sc as plsc

_V = 100000
_H = 128
_VBLK = 10240


def _gather_rows(table, idx, n_tok):
    info = plsc.get_sparse_core_info()
    nw = info.num_cores * info.num_subcores
    per = n_tok // nw
    mesh = plsc.VectorSubcoreMesh(core_axis_name="c", subcore_axis_name="s")

    @functools.partial(
        pl.kernel,
        mesh=mesh,
        out_type=jax.ShapeDtypeStruct((n_tok, _H), jnp.float32),
        scratch_types=[
            pltpu.VMEM((per,), jnp.int32),
            pltpu.VMEM((per, _H), jnp.float32),
            pltpu.SemaphoreType.DMA,
        ],
    )
    def k(table_hbm, idx_hbm, out_hbm, idx_v, rows_v, sem):
        wid = lax.axis_index("s") * info.num_cores + lax.axis_index("c")
        base = wid * per
        pltpu.sync_copy(idx_hbm.at[pl.ds(base, per)], idx_v)
        pltpu.async_copy(table_hbm.at[idx_v], rows_v, sem).wait()
        pltpu.sync_copy(rows_v, out_hbm.at[pl.ds(base, per)])

    return k(table, idx)


def _mm_body(h_ref, w_ref, o_ref):
    o_ref[...] = lax.dot_general(
        h_ref[...],
        w_ref[...],
        (((1,), (1,)), ((), ())),
        preferred_element_type=jnp.float32,
    )


def kernel(input_ids, wte, lm_head_w):
    b, s = input_ids.shape
    n_tok = b * s
    idx = input_ids.reshape(n_tok).astype(jnp.int32)
    hidden = _gather_rows(wte, idx, n_tok)
    nblk = pl.cdiv(_V, _VBLK)
    logits = pl.pallas_call(
        _mm_body,
        grid=(nblk,),
        in_specs=[
            pl.BlockSpec((n_tok, _H), lambda i: (0, 0)),
            pl.BlockSpec((_VBLK, _H), lambda i: (i, 0)),
        ],
        out_specs=pl.BlockSpec((n_tok, _VBLK), lambda i: (0, i)),
        out_shape=jax.ShapeDtypeStruct((n_tok, _V), jnp.float32),
        compiler_params=pltpu.CompilerParams(
            dimension_semantics=("arbitrary",),
        ),
    )(hidden, lm_head_w)
    return logits.reshape(b, s, _V)
